# baseline (speedup 1.0000x reference)
import functools
import jax
import jax.numpy as jnp
from jax import lax
from jax.experimental import pallas as pl
from jax.experimental.pallas import tpu as pltpu
from jax.experimental.pallas import tpu_sc as plsc

N = 10000
E = 320000
D = 128
H = 128
B = 4096
R = 5

NC = 2
NS = 16
NW = NC * NS
CHUNK = 128
ROWS = E // CHUNK
NITER = 80
ROWSP = NW * NITER
NBUF = 2
RPT = 632
NP = NS * RPT
DA = 144

_mesh = functools.partial(
    plsc.VectorSubcoreMesh, core_axis_name="c", subcore_axis_name="s",
    num_cores=NC, num_subcores=NS)


def _sc_scatter(feats, src2d, dst2d, zeros_f, zeros_d, ones16):
  f32 = jnp.float32
  NB = 2
  NR = NITER // NB

  @functools.partial(
      pl.kernel,
      out_type=(jax.ShapeDtypeStruct((NC * NP, H), f32),
                jax.ShapeDtypeStruct((NC * NP, 16), f32)),
      mesh=_mesh(),
      scratch_types=[
          [pltpu.VMEM((CHUNK,), jnp.int32) for _ in range(NB)],
          [pltpu.VMEM((CHUNK,), jnp.int32) for _ in range(NB)],
          [pltpu.VMEM((CHUNK, H), f32) for _ in range(NB)],
          pltpu.VMEM((CHUNK, 16), f32),
          pltpu.VMEM_SHARED((NP, H), f32),
          pltpu.VMEM_SHARED((NP, 16), f32),
          [pltpu.SemaphoreType.DMA for _ in range(NB)],
          [pltpu.SemaphoreType.DMA for _ in range(NB)],
          [pltpu.SemaphoreType.DMA for _ in range(NB)],
          [pltpu.SemaphoreType.DMA for _ in range(NB)],
      ],
      compiler_params=pltpu.CompilerParams(use_tc_tiling_on_sc=False),
  )
  def k(feats_h, src_h, dst_h, zf_h, zd_h, ones_h, out_h, dout_h,
        sbuf, dbuf, buf, ones16v, acc, dacc, gsem, ssem, issem, idsem):
    c = lax.axis_index("c")
    s = lax.axis_index("s")
    wid = s * NC + c
    base = wid * NITER

    pltpu.sync_copy(ones_h, ones16v)
    pltpu.sync_copy(zf_h, acc.at[pl.ds(s * RPT, RPT)])
    pltpu.sync_copy(zd_h, dacc.at[pl.ds(s * RPT, RPT)])
    plsc.subcore_barrier()

    for b in range(NB):
      pltpu.sync_copy(src_h.at[base + b], sbuf[b])
      pltpu.sync_copy(dst_h.at[base + b], dbuf[b])
      pltpu.async_copy(feats_h.at[sbuf[b]], buf[b], gsem[b])

    def body(j, carry):
      for b in range(NB):
        i = j * NB + b
        pltpu.make_async_copy(feats_h.at[sbuf[b]], buf[b], gsem[b]).wait()

        @pl.when(j > 0)
        def _():
          pltpu.make_async_copy(dst_h.at[0], dbuf[b], idsem[b]).wait()

        pltpu.async_copy(buf[b], acc.at[dbuf[b]], ssem[b], add=True)
        pltpu.sync_copy(ones16v, dacc.at[dbuf[b]], add=True)

        @pl.when(j < NR - 1)
        def _():
          pltpu.async_copy(src_h.at[base + i + NB], sbuf[b], issem[b])
          pltpu.make_async_copy(buf[b], acc.at[dbuf[b]], ssem[b]).wait()
          pltpu.async_copy(dst_h.at[base + i + NB], dbuf[b], idsem[b])
          pltpu.make_async_copy(src_h.at[0], sbuf[b], issem[b]).wait()
          pltpu.async_copy(feats_h.at[sbuf[b]], buf[b], gsem[b])

        @pl.when(j == NR - 1)
        def _():
          pltpu.make_async_copy(buf[b], acc.at[dbuf[b]], ssem[b]).wait()

      return carry

    lax.fori_loop(0, NR, body, 0)
    plsc.subcore_barrier()

    pltpu.sync_copy(acc.at[pl.ds(s * RPT, RPT)],
                    out_h.at[pl.ds(c * NP + s * RPT, RPT)])
    pltpu.sync_copy(dacc.at[pl.ds(s * RPT, RPT)],
                    dout_h.at[pl.ds(c * NP + s * RPT, RPT)])

  return k(feats, src2d, dst2d, zeros_f, zeros_d, ones16)


def _sc_gather(uids2d, s2a, s2b, h1, x, inv16):
  f32 = jnp.float32
  outs = (
      jax.ShapeDtypeStruct((B, 128), f32),
      jax.ShapeDtypeStruct((B, 128), f32),
      jax.ShapeDtypeStruct((B, 128), f32),
      jax.ShapeDtypeStruct((B, 128), f32),
      jax.ShapeDtypeStruct((B, 16), f32),
  )

  @functools.partial(
      pl.kernel,
      out_type=outs,
      mesh=_mesh(),
      scratch_types=[
          pltpu.VMEM((CHUNK,), jnp.int32),
          pltpu.VMEM((CHUNK, 128), jnp.float32),
          pltpu.VMEM((CHUNK, 16), jnp.float32),
          pltpu.SemaphoreType.DMA,
      ],
      compiler_params=pltpu.CompilerParams(use_tc_tiling_on_sc=False),
  )
  def k(uids_h, a_h, b_h, h1_h, x_h, inv_h,
        oa_h, ob_h, oh_h, ox_h, oi_h, uid_v, buf, buf16, sem):
    c = lax.axis_index("c")
    s = lax.axis_index("s")
    wid = s * NC + c
    pltpu.sync_copy(uids_h.at[wid], uid_v)
    for src_h, dst_h in ((a_h, oa_h), (b_h, ob_h), (h1_h, oh_h), (x_h, ox_h)):
      pltpu.async_copy(src_h.at[uid_v], buf, sem).wait()
      pltpu.sync_copy(buf, dst_h.at[pl.ds(wid * CHUNK, CHUNK)])
    pltpu.async_copy(inv_h.at[uid_v], buf16, sem).wait()
    pltpu.sync_copy(buf16, oi_h.at[pl.ds(wid * CHUNK, CHUNK)])

  return k(uids2d, s2a, s2b, h1, x, inv16)


def _tc_layer1(sa, sb, da, db, x, wl1t, bl1, wr1t):
  BLK = 1000
  f32 = jnp.float32

  def body(sa_r, sb_r, da_r, db_r, x_r, wl_r, bl_r, wr_r, h1_r, inv_r):
    s = sa_r[...] + sb_r[...]
    deg = jnp.maximum(da_r[...][:, 0:1] + db_r[...][:, 0:1], 1.0)
    inv = 1.0 / deg
    m = s * inv
    h = (jnp.dot(m, wl_r[...], preferred_element_type=f32) + bl_r[...]
         + jnp.dot(x_r[...], wr_r[...], preferred_element_type=f32))
    h1_r[...] = jnp.maximum(h, 0.0)
    inv_r[...] = jnp.broadcast_to(inv, (BLK, 16))

  blk = lambda m, n: pl.BlockSpec((m, n), lambda i: (i, 0))
  whole = lambda m, n: pl.BlockSpec((m, n), lambda i: (0, 0))
  return pl.pallas_call(
      body,
      grid=(N // BLK,),
      in_specs=[blk(BLK, 128), blk(BLK, 128), blk(BLK, 16), blk(BLK, 16),
                blk(BLK, 128), whole(128, 128), whole(1, 128),
                whole(128, 128)],
      out_specs=[blk(BLK, 128), blk(BLK, 16)],
      out_shape=[jax.ShapeDtypeStruct((N, 128), f32),
                 jax.ShapeDtypeStruct((N, 16), f32)],
  )(sa, sb, da, db, x, wl1t, bl1, wr1t)


def _tc_epilogue(ua, ub, uh1, ux, uinv, roh8,
                 wl2t, bl2, wr2t, wiha, wihb, wihr8, bih,
                 wc1at, wc1bt, bc1, wc2t, bc2, wc3t8, bc3p):
  BLK = 512
  f32 = jnp.float32

  def body(ua_r, ub_r, uh1_r, ux_r, uinv_r, roh_r,
           wl2_r, bl2_r, wr2_r, wiha_r, wihb_r, wihr_r, bih_r,
           wc1a_r, wc1b_r, bc1_r, wc2_r, bc2_r, wc3_r, bc3_r, out_r):
    dot = lambda a, b: jnp.dot(a, b, preferred_element_type=f32)
    m2 = (ua_r[...] + ub_r[...]) * uinv_r[...][:, 0:1]
    ue = dot(m2, wl2_r[...]) + bl2_r[...] + dot(uh1_r[...], wr2_r[...])
    ue = jnp.clip(ue, -10.0, 10.0)
    uf = jnp.clip(ux_r[...], -10.0, 10.0)
    gates = (dot(ue, wiha_r[...]) + dot(uf, wihb_r[...])
             + dot(roh_r[...], wihr_r[...]) + bih_r[...])
    i_g = gates[:, 0:128]
    g_g = gates[:, 256:384]
    o_g = gates[:, 384:512]
    cc = jax.nn.sigmoid(i_g) * jnp.tanh(g_g)
    lo = jnp.clip(jax.nn.sigmoid(o_g) * jnp.tanh(cc), -10.0, 10.0)
    z = jnp.maximum(dot(ue, wc1a_r[...]) + dot(lo, wc1b_r[...]) + bc1_r[...],
                    0.0)
    z2 = jnp.maximum(dot(z, wc2_r[...]) + bc2_r[...], 0.0)
    out_r[...] = dot(z2, wc3_r[...]) + bc3_r[...]

  blk = lambda m, n: pl.BlockSpec((m, n), lambda i: (i, 0))
  whole = lambda m, n: pl.BlockSpec((m, n), lambda i: (0, 0))
  return pl.pallas_call(
      body,
      grid=(B // BLK,),
      in_specs=[blk(BLK, 128), blk(BLK, 128), blk(BLK, 128), blk(BLK, 128),
                blk(BLK, 16), blk(BLK, 8),
                whole(128, 128), whole(1, 128), whole(128, 128),
                whole(128, 512), whole(128, 512), whole(8, 512),
                whole(1, 512),
                whole(128, 128), whole(128, 128), whole(1, 128),
                whole(128, 64), whole(1, 64), whole(64, 8), whole(1, 8)],
      out_specs=blk(BLK, 8),
      out_shape=jax.ShapeDtypeStruct((B, 8), f32),
  )(ua, ub, uh1, ux, uinv, roh8,
    wl2t, bl2, wr2t, wiha, wihb, wihr8, bih,
    wc1at, wc1bt, bc1, wc2t, bc2, wc3t8, bc3p)


def kernel(x, edge_index, user_ids, current_roles,
           Wl1, bl1, Wr1, Wl2, bl2, Wr2,
           W_ih, W_hh, b_ih, b_hh,
           Wc1, bc1, Wc2, bc2, Wc3, bc3):
  f32 = jnp.float32
  pad_rows = ROWSP - ROWS
  src2d = jnp.concatenate(
      [edge_index[0].reshape(ROWS, CHUNK),
       jnp.zeros((pad_rows, CHUNK), jnp.int32)], axis=0)
  dst2d = jnp.concatenate(
      [edge_index[1].reshape(ROWS, CHUNK),
       jnp.full((pad_rows, CHUNK), NP - 1, jnp.int32)], axis=0)
  uids2d = user_ids.reshape(NW, CHUNK)

  zeros_f = jnp.zeros((RPT, H), f32)
  zeros_d = jnp.zeros((RPT, 16), f32)
  ones16 = jnp.zeros((CHUNK, 16), f32).at[:, 0].set(1.0)

  s1, d1 = _sc_scatter(x, src2d, dst2d, zeros_f, zeros_d, ones16)
  s1a, s1b = s1[:N], s1[NP:NP + N]

  h1, inv16 = _tc_layer1(
      s1a, s1b, d1[:N], d1[NP:NP + N],
      x, Wl1.T, bl1.reshape(1, H), Wr1.T)

  s2, _unused_d2 = _sc_scatter(h1, src2d, dst2d, zeros_f, zeros_d, ones16)
  s2a, s2b = s2[:N], s2[NP:NP + N]

  ua, ub, uh1, ux, uinv = _sc_gather(uids2d, s2a, s2b, h1, x, inv16)

  roh8 = jax.nn.one_hot(current_roles, 8, dtype=f32)
  wihr8 = jnp.zeros((8, 4 * H), f32).at[:R].set(W_ih[:, 2 * H:].T)
  wc3t8 = jnp.zeros((H // 2, 8), f32).at[:, :R].set(Wc3.T)
  bc3p = jnp.zeros((1, 8), f32).at[:, :R].set(bc3)

  out8 = _tc_epilogue(
      ua, ub, uh1, ux, uinv, roh8,
      Wl2.T, bl2.reshape(1, H), Wr2.T,
      W_ih[:, :H].T, W_ih[:, H:2 * H].T, wihr8,
      (b_ih + b_hh).reshape(1, 4 * H),
      Wc1[:, :H].T, Wc1[:, H:].T, bc1.reshape(1, H),
      Wc2.T, bc2.reshape(1, H // 2), wc3t8, bc3p)
  return out8[:, :R]


# speedup vs baseline: 1.1563x; 1.1563x over previous
import functools
import jax
import jax.numpy as jnp
from jax import lax
from jax.experimental import pallas as pl
from jax.experimental.pallas import tpu as pltpu
from jax.experimental.pallas import tpu_sc as plsc

N = 10000
E = 320000
D = 128
H = 128
B = 4096
R = 5

NC = 2
NS = 16
NW = NC * NS
CHUNK = 128
ROWS = E // CHUNK
NITER = 80
ROWSP = NW * NITER
NBUF = 2
RPT = 632
NP = NS * RPT
DA = 144

_mesh = functools.partial(
    plsc.VectorSubcoreMesh, core_axis_name="c", subcore_axis_name="s",
    num_cores=NC, num_subcores=NS)


def _sc_scatter(feats, src2d, dst2d, zeros_f, zeros_d, ones16, with_deg):
  f32 = jnp.float32
  NB = 2
  NR = NITER // NB

  @functools.partial(
      pl.kernel,
      out_type=(jax.ShapeDtypeStruct((NC * NP, H), f32),
                jax.ShapeDtypeStruct((NC * NP, 16), f32)),
      mesh=_mesh(),
      scratch_types=[
          [pltpu.VMEM((CHUNK,), jnp.int32) for _ in range(NB)],
          [pltpu.VMEM((CHUNK,), jnp.int32) for _ in range(NB)],
          [pltpu.VMEM((CHUNK, H), f32) for _ in range(NB)],
          pltpu.VMEM((CHUNK, 16), f32),
          pltpu.VMEM_SHARED((NP, H), f32),
          pltpu.VMEM_SHARED((NP, 16), f32),
          [pltpu.SemaphoreType.DMA for _ in range(NB)],
          [pltpu.SemaphoreType.DMA for _ in range(NB)],
          [pltpu.SemaphoreType.DMA for _ in range(NB)],
          [pltpu.SemaphoreType.DMA for _ in range(NB)],
      ],
      compiler_params=pltpu.CompilerParams(use_tc_tiling_on_sc=False),
  )
  def k(feats_h, src_h, dst_h, zf_h, zd_h, ones_h, out_h, dout_h,
        sbuf, dbuf, buf, ones16v, acc, dacc, gsem, ssem, issem, idsem):
    c = lax.axis_index("c")
    s = lax.axis_index("s")
    wid = s * NC + c
    base = wid * NITER

    pltpu.sync_copy(ones_h, ones16v)
    pltpu.sync_copy(zf_h, acc.at[pl.ds(s * RPT, RPT)])
    pltpu.sync_copy(zd_h, dacc.at[pl.ds(s * RPT, RPT)])
    plsc.subcore_barrier()

    for b in range(NB):
      pltpu.sync_copy(src_h.at[base + b], sbuf[b])
      pltpu.sync_copy(dst_h.at[base + b], dbuf[b])
      pltpu.async_copy(feats_h.at[sbuf[b]], buf[b], gsem[b])

    def body(j, carry):
      for b in range(NB):
        i = j * NB + b
        pltpu.make_async_copy(feats_h.at[sbuf[b]], buf[b], gsem[b]).wait()

        @pl.when(j > 0)
        def _():
          pltpu.make_async_copy(dst_h.at[0], dbuf[b], idsem[b]).wait()

        pltpu.async_copy(buf[b], acc.at[dbuf[b]], ssem[b], add=True)
        if with_deg:
          pltpu.sync_copy(ones16v, dacc.at[dbuf[b]], add=True)

        @pl.when(j < NR - 1)
        def _():
          pltpu.async_copy(src_h.at[base + i + NB], sbuf[b], issem[b])
          pltpu.make_async_copy(buf[b], acc.at[dbuf[b]], ssem[b]).wait()
          pltpu.async_copy(dst_h.at[base + i + NB], dbuf[b], idsem[b])
          pltpu.make_async_copy(src_h.at[0], sbuf[b], issem[b]).wait()
          pltpu.async_copy(feats_h.at[sbuf[b]], buf[b], gsem[b])

        @pl.when(j == NR - 1)
        def _():
          pltpu.make_async_copy(buf[b], acc.at[dbuf[b]], ssem[b]).wait()

      return carry

    lax.fori_loop(0, NR, body, 0)
    plsc.subcore_barrier()

    pltpu.sync_copy(acc.at[pl.ds(s * RPT, RPT)],
                    out_h.at[pl.ds(c * NP + s * RPT, RPT)])
    pltpu.sync_copy(dacc.at[pl.ds(s * RPT, RPT)],
                    dout_h.at[pl.ds(c * NP + s * RPT, RPT)])

  return k(feats, src2d, dst2d, zeros_f, zeros_d, ones16)


def _chunk_row_map():
  import numpy as np
  w = np.arange(NW)[:, None]
  kk = np.arange(NITER)[None, :]
  nreal = 78 + (w < 4)
  real = w * 78 + np.minimum(w, 4) + kk
  return jnp.asarray(np.where(kk < nreal, real, ROWS).reshape(-1),
                     dtype=jnp.int32)


def _sc_gather(uids2d, s2a, s2b, h1, x, inv16):
  f32 = jnp.float32
  outs = (
      jax.ShapeDtypeStruct((B, 128), f32),
      jax.ShapeDtypeStruct((B, 128), f32),
      jax.ShapeDtypeStruct((B, 128), f32),
      jax.ShapeDtypeStruct((B, 128), f32),
      jax.ShapeDtypeStruct((B, 16), f32),
  )

  @functools.partial(
      pl.kernel,
      out_type=outs,
      mesh=_mesh(),
      scratch_types=[
          pltpu.VMEM((CHUNK,), jnp.int32),
          pltpu.VMEM((CHUNK, 128), jnp.float32),
          pltpu.VMEM((CHUNK, 16), jnp.float32),
          pltpu.SemaphoreType.DMA,
      ],
      compiler_params=pltpu.CompilerParams(use_tc_tiling_on_sc=False),
  )
  def k(uids_h, a_h, b_h, h1_h, x_h, inv_h,
        oa_h, ob_h, oh_h, ox_h, oi_h, uid_v, buf, buf16, sem):
    c = lax.axis_index("c")
    s = lax.axis_index("s")
    wid = s * NC + c
    pltpu.sync_copy(uids_h.at[wid], uid_v)
    for src_h, dst_h in ((a_h, oa_h), (b_h, ob_h), (h1_h, oh_h), (x_h, ox_h)):
      pltpu.async_copy(src_h.at[uid_v], buf, sem).wait()
      pltpu.sync_copy(buf, dst_h.at[pl.ds(wid * CHUNK, CHUNK)])
    pltpu.async_copy(inv_h.at[uid_v], buf16, sem).wait()
    pltpu.sync_copy(buf16, oi_h.at[pl.ds(wid * CHUNK, CHUNK)])

  return k(uids2d, s2a, s2b, h1, x, inv16)


def _tc_layer1(sa, sb, da, db, x, wl1t, bl1, wr1t):
  BLK = 1000
  f32 = jnp.float32

  def body(sa_r, sb_r, da_r, db_r, x_r, wl_r, bl_r, wr_r, h1_r, inv_r):
    s = sa_r[...] + sb_r[...]
    deg = jnp.maximum(da_r[...][:, 0:1] + db_r[...][:, 0:1], 1.0)
    inv = 1.0 / deg
    m = s * inv
    h = (jnp.dot(m, wl_r[...], preferred_element_type=f32) + bl_r[...]
         + jnp.dot(x_r[...], wr_r[...], preferred_element_type=f32))
    h1_r[...] = jnp.maximum(h, 0.0)
    inv_r[...] = jnp.broadcast_to(inv, (BLK, 16))

  blk = lambda m, n: pl.BlockSpec((m, n), lambda i: (i, 0))
  whole = lambda m, n: pl.BlockSpec((m, n), lambda i: (0, 0))
  return pl.pallas_call(
      body,
      grid=(N // BLK,),
      in_specs=[blk(BLK, 128), blk(BLK, 128), blk(BLK, 16), blk(BLK, 16),
                blk(BLK, 128), whole(128, 128), whole(1, 128),
                whole(128, 128)],
      out_specs=[blk(BLK, 128), blk(BLK, 16)],
      out_shape=[jax.ShapeDtypeStruct((N, 128), f32),
                 jax.ShapeDtypeStruct((N, 16), f32)],
  )(sa, sb, da, db, x, wl1t, bl1, wr1t)


def _tc_epilogue(ua, ub, uh1, ux, uinv, roh8,
                 wl2t, bl2, wr2t, wiha, wihb, wihr8, bih,
                 wc1at, wc1bt, bc1, wc2t, bc2, wc3t8, bc3p):
  BLK = 512
  f32 = jnp.float32

  def body(ua_r, ub_r, uh1_r, ux_r, uinv_r, roh_r,
           wl2_r, bl2_r, wr2_r, wiha_r, wihb_r, wihr_r, bih_r,
           wc1a_r, wc1b_r, bc1_r, wc2_r, bc2_r, wc3_r, bc3_r, out_r):
    dot = lambda a, b: jnp.dot(a, b, preferred_element_type=f32)
    m2 = (ua_r[...] + ub_r[...]) * uinv_r[...][:, 0:1]
    ue = dot(m2, wl2_r[...]) + bl2_r[...] + dot(uh1_r[...], wr2_r[...])
    ue = jnp.clip(ue, -10.0, 10.0)
    uf = jnp.clip(ux_r[...], -10.0, 10.0)
    gates = (dot(ue, wiha_r[...]) + dot(uf, wihb_r[...])
             + dot(roh_r[...], wihr_r[...]) + bih_r[...])
    i_g = gates[:, 0:128]
    g_g = gates[:, 256:384]
    o_g = gates[:, 384:512]
    cc = jax.nn.sigmoid(i_g) * jnp.tanh(g_g)
    lo = jnp.clip(jax.nn.sigmoid(o_g) * jnp.tanh(cc), -10.0, 10.0)
    z = jnp.maximum(dot(ue, wc1a_r[...]) + dot(lo, wc1b_r[...]) + bc1_r[...],
                    0.0)
    z2 = jnp.maximum(dot(z, wc2_r[...]) + bc2_r[...], 0.0)
    out_r[...] = dot(z2, wc3_r[...]) + bc3_r[...]

  blk = lambda m, n: pl.BlockSpec((m, n), lambda i: (i, 0))
  whole = lambda m, n: pl.BlockSpec((m, n), lambda i: (0, 0))
  return pl.pallas_call(
      body,
      grid=(B // BLK,),
      in_specs=[blk(BLK, 128), blk(BLK, 128), blk(BLK, 128), blk(BLK, 128),
                blk(BLK, 16), blk(BLK, 8),
                whole(128, 128), whole(1, 128), whole(128, 128),
                whole(128, 512), whole(128, 512), whole(8, 512),
                whole(1, 512),
                whole(128, 128), whole(128, 128), whole(1, 128),
                whole(128, 64), whole(1, 64), whole(64, 8), whole(1, 8)],
      out_specs=blk(BLK, 8),
      out_shape=jax.ShapeDtypeStruct((B, 8), f32),
  )(ua, ub, uh1, ux, uinv, roh8,
    wl2t, bl2, wr2t, wiha, wihb, wihr8, bih,
    wc1at, wc1bt, bc1, wc2t, bc2, wc3t8, bc3p)


def kernel(x, edge_index, user_ids, current_roles,
           Wl1, bl1, Wr1, Wl2, bl2, Wr2,
           W_ih, W_hh, b_ih, b_hh,
           Wc1, bc1, Wc2, bc2, Wc3, bc3):
  f32 = jnp.float32
  rmap = _chunk_row_map()
  pad_src = jnp.zeros((1, CHUNK), jnp.int32)
  pad_dst = (N + (jnp.arange(CHUNK, dtype=jnp.int32) % (NP - N))).reshape(
      1, CHUNK)
  src2d = jnp.take(
      jnp.concatenate([edge_index[0].reshape(ROWS, CHUNK), pad_src], axis=0),
      rmap, axis=0)
  dst2d = jnp.take(
      jnp.concatenate([edge_index[1].reshape(ROWS, CHUNK), pad_dst], axis=0),
      rmap, axis=0)
  uids2d = user_ids.reshape(NW, CHUNK)

  zeros_f = jnp.zeros((RPT, H), f32)
  zeros_d = jnp.zeros((RPT, 16), f32)
  ones16 = jnp.zeros((CHUNK, 16), f32).at[:, 0].set(1.0)

  s1, d1 = _sc_scatter(x, src2d, dst2d, zeros_f, zeros_d, ones16, True)
  s1a, s1b = s1[:N], s1[NP:NP + N]

  h1, inv16 = _tc_layer1(
      s1a, s1b, d1[:N], d1[NP:NP + N],
      x, Wl1.T, bl1.reshape(1, H), Wr1.T)

  s2, _unused_d2 = _sc_scatter(h1, src2d, dst2d, zeros_f, zeros_d, ones16,
                               False)
  s2a, s2b = s2[:N], s2[NP:NP + N]

  ua, ub, uh1, ux, uinv = _sc_gather(uids2d, s2a, s2b, h1, x, inv16)

  roh8 = jax.nn.one_hot(current_roles, 8, dtype=f32)
  wihr8 = jnp.zeros((8, 4 * H), f32).at[:R].set(W_ih[:, 2 * H:].T)
  wc3t8 = jnp.zeros((H // 2, 8), f32).at[:, :R].set(Wc3.T)
  bc3p = jnp.zeros((1, 8), f32).at[:, :R].set(bc3)

  out8 = _tc_epilogue(
      ua, ub, uh1, ux, uinv, roh8,
      Wl2.T, bl2.reshape(1, H), Wr2.T,
      W_ih[:, :H].T, W_ih[:, H:2 * H].T, wihr8,
      (b_ih + b_hh).reshape(1, 4 * H),
      Wc1[:, :H].T, Wc1[:, H:].T, bc1.reshape(1, H),
      Wc2.T, bc2.reshape(1, H // 2), wc3t8, bc3p)
  return out8[:, :R]


# speedup vs baseline: 3.1596x; 2.7326x over previous
import functools
import jax
import jax.numpy as jnp
from jax import lax
from jax.experimental import pallas as pl
from jax.experimental.pallas import tpu as pltpu
from jax.experimental.pallas import tpu_sc as plsc

N = 10000
E = 320000
D = 128
H = 128
B = 4096
R = 5

NC = 2
NS = 16
NW = NC * NS
CHUNK = 128
ROWS = E // CHUNK
NITER = 80
ROWSP = NW * NITER
NBUF = 2
RPT = 632
NP = NS * RPT
DA = 144

_mesh = functools.partial(
    plsc.VectorSubcoreMesh, core_axis_name="c", subcore_axis_name="s",
    num_cores=NC, num_subcores=NS)


def _sc_scatter(feats, src2d, dst2d, zeros_f, zeros_d, ones16, with_deg):
  f32 = jnp.float32
  NB = 2
  NR = NITER // NB

  @functools.partial(
      pl.kernel,
      out_type=(jax.ShapeDtypeStruct((NC * NP, H), f32),
                jax.ShapeDtypeStruct((NC * NP, 16), f32)),
      mesh=_mesh(),
      scratch_types=[
          [pltpu.VMEM((CHUNK,), jnp.int32) for _ in range(NB)],
          [pltpu.VMEM((CHUNK,), jnp.int32) for _ in range(NB)],
          [pltpu.VMEM((CHUNK, H), f32) for _ in range(NB)],
          pltpu.VMEM((CHUNK, 16), f32),
          pltpu.VMEM_SHARED((NP, H), f32),
          pltpu.VMEM_SHARED((NP, 16), f32),
          [pltpu.SemaphoreType.DMA for _ in range(NB)],
          [pltpu.SemaphoreType.DMA for _ in range(NB)],
          [pltpu.SemaphoreType.DMA for _ in range(NB)],
          [pltpu.SemaphoreType.DMA for _ in range(NB)],
      ],
      compiler_params=pltpu.CompilerParams(use_tc_tiling_on_sc=False),
  )
  def k(feats_h, src_h, dst_h, zf_h, zd_h, ones_h, out_h, dout_h,
        sbuf, dbuf, buf, ones16v, acc, dacc, gsem, ssem, issem, idsem):
    c = lax.axis_index("c")
    s = lax.axis_index("s")
    wid = s * NC + c
    base = 78 * wid + jnp.minimum(wid, 4)
    nreal = 78 + jnp.where(wid < 4, 1, 0)

    pltpu.sync_copy(ones_h, ones16v)
    pltpu.sync_copy(zf_h, acc.at[pl.ds(s * RPT, RPT)])
    pltpu.sync_copy(zd_h, dacc.at[pl.ds(s * RPT, RPT)])
    plsc.subcore_barrier()

    for b in range(NB):
      pltpu.sync_copy(src_h.at[base + b], sbuf[b])
      pltpu.sync_copy(dst_h.at[base + b], dbuf[b])
      pltpu.async_copy(feats_h.at[sbuf[b]], buf[b], gsem[b])

    def body(j, carry):
      for b in range(NB):
        i = j * NB + b
        valid = i < nreal
        nxt_valid = (i + NB) < nreal

        @pl.when(valid)
        def _():
          pltpu.make_async_copy(feats_h.at[sbuf[b]], buf[b], gsem[b]).wait()

          @pl.when(j > 0)
          def _():
            pltpu.make_async_copy(dst_h.at[0], dbuf[b], idsem[b]).wait()

          pltpu.async_copy(buf[b], acc.at[dbuf[b]], ssem[b], add=True)
          if with_deg:
            pltpu.sync_copy(ones16v, dacc.at[dbuf[b]], add=True)

        @pl.when(nxt_valid)
        def _():
          pltpu.async_copy(src_h.at[base + i + NB], sbuf[b], issem[b])

        @pl.when(valid)
        def _():
          pltpu.make_async_copy(buf[b], acc.at[dbuf[b]], ssem[b]).wait()

        @pl.when(nxt_valid)
        def _():
          pltpu.async_copy(dst_h.at[base + i + NB], dbuf[b], idsem[b])
          pltpu.make_async_copy(src_h.at[0], sbuf[b], issem[b]).wait()
          pltpu.async_copy(feats_h.at[sbuf[b]], buf[b], gsem[b])

      return carry

    lax.fori_loop(0, NR, body, 0)
    plsc.subcore_barrier()

    pltpu.sync_copy(acc.at[pl.ds(s * RPT, RPT)],
                    out_h.at[pl.ds(c * NP + s * RPT, RPT)])
    pltpu.sync_copy(dacc.at[pl.ds(s * RPT, RPT)],
                    dout_h.at[pl.ds(c * NP + s * RPT, RPT)])

  return k(feats, src2d, dst2d, zeros_f, zeros_d, ones16)


def _sc_gather(uids2d, s2a, s2b, h1, x, inv16):
  f32 = jnp.float32
  outs = (
      jax.ShapeDtypeStruct((B, 128), f32),
      jax.ShapeDtypeStruct((B, 128), f32),
      jax.ShapeDtypeStruct((B, 128), f32),
      jax.ShapeDtypeStruct((B, 128), f32),
      jax.ShapeDtypeStruct((B, 16), f32),
  )

  @functools.partial(
      pl.kernel,
      out_type=outs,
      mesh=_mesh(),
      scratch_types=[
          pltpu.VMEM((CHUNK,), jnp.int32),
          pltpu.VMEM((CHUNK, 128), jnp.float32),
          pltpu.VMEM((CHUNK, 16), jnp.float32),
          pltpu.SemaphoreType.DMA,
      ],
      compiler_params=pltpu.CompilerParams(use_tc_tiling_on_sc=False),
  )
  def k(uids_h, a_h, b_h, h1_h, x_h, inv_h,
        oa_h, ob_h, oh_h, ox_h, oi_h, uid_v, buf, buf16, sem):
    c = lax.axis_index("c")
    s = lax.axis_index("s")
    wid = s * NC + c
    pltpu.sync_copy(uids_h.at[wid], uid_v)
    for src_h, dst_h in ((a_h, oa_h), (b_h, ob_h), (h1_h, oh_h), (x_h, ox_h)):
      pltpu.async_copy(src_h.at[uid_v], buf, sem).wait()
      pltpu.sync_copy(buf, dst_h.at[pl.ds(wid * CHUNK, CHUNK)])
    pltpu.async_copy(inv_h.at[uid_v], buf16, sem).wait()
    pltpu.sync_copy(buf16, oi_h.at[pl.ds(wid * CHUNK, CHUNK)])

  return k(uids2d, s2a, s2b, h1, x, inv16)


def _tc_layer1(sa, sb, da, db, x, wl1t, bl1, wr1t):
  BLK = 1000
  f32 = jnp.float32

  def body(sa_r, sb_r, da_r, db_r, x_r, wl_r, bl_r, wr_r, h1_r, inv_r):
    s = sa_r[...] + sb_r[...]
    deg = jnp.maximum(da_r[...][:, 0:1] + db_r[...][:, 0:1], 1.0)
    inv = 1.0 / deg
    m = s * inv
    h = (jnp.dot(m, wl_r[...], preferred_element_type=f32) + bl_r[...]
         + jnp.dot(x_r[...], wr_r[...], preferred_element_type=f32))
    h1_r[...] = jnp.maximum(h, 0.0)
    inv_r[...] = jnp.broadcast_to(inv, (BLK, 16))

  blk = lambda m, n: pl.BlockSpec((m, n), lambda i: (i, 0))
  whole = lambda m, n: pl.BlockSpec((m, n), lambda i: (0, 0))
  return pl.pallas_call(
      body,
      grid=(N // BLK,),
      in_specs=[blk(BLK, 128), blk(BLK, 128), blk(BLK, 16), blk(BLK, 16),
                blk(BLK, 128), whole(128, 128), whole(1, 128),
                whole(128, 128)],
      out_specs=[blk(BLK, 128), blk(BLK, 16)],
      out_shape=[jax.ShapeDtypeStruct((N, 128), f32),
                 jax.ShapeDtypeStruct((N, 16), f32)],
  )(sa, sb, da, db, x, wl1t, bl1, wr1t)


def _tc_epilogue(ua, ub, uh1, ux, uinv, roh8,
                 wl2t, bl2, wr2t, wiha, wihb, wihr8, bih,
                 wc1at, wc1bt, bc1, wc2t, bc2, wc3t8, bc3p):
  BLK = 512
  f32 = jnp.float32

  def body(ua_r, ub_r, uh1_r, ux_r, uinv_r, roh_r,
           wl2_r, bl2_r, wr2_r, wiha_r, wihb_r, wihr_r, bih_r,
           wc1a_r, wc1b_r, bc1_r, wc2_r, bc2_r, wc3_r, bc3_r, out_r):
    dot = lambda a, b: jnp.dot(a, b, preferred_element_type=f32)
    m2 = (ua_r[...] + ub_r[...]) * uinv_r[...][:, 0:1]
    ue = dot(m2, wl2_r[...]) + bl2_r[...] + dot(uh1_r[...], wr2_r[...])
    ue = jnp.clip(ue, -10.0, 10.0)
    uf = jnp.clip(ux_r[...], -10.0, 10.0)
    gates = (dot(ue, wiha_r[...]) + dot(uf, wihb_r[...])
             + dot(roh_r[...], wihr_r[...]) + bih_r[...])
    i_g = gates[:, 0:128]
    g_g = gates[:, 256:384]
    o_g = gates[:, 384:512]
    cc = jax.nn.sigmoid(i_g) * jnp.tanh(g_g)
    lo = jnp.clip(jax.nn.sigmoid(o_g) * jnp.tanh(cc), -10.0, 10.0)
    z = jnp.maximum(dot(ue, wc1a_r[...]) + dot(lo, wc1b_r[...]) + bc1_r[...],
                    0.0)
    z2 = jnp.maximum(dot(z, wc2_r[...]) + bc2_r[...], 0.0)
    out_r[...] = dot(z2, wc3_r[...]) + bc3_r[...]

  blk = lambda m, n: pl.BlockSpec((m, n), lambda i: (i, 0))
  whole = lambda m, n: pl.BlockSpec((m, n), lambda i: (0, 0))
  return pl.pallas_call(
      body,
      grid=(B // BLK,),
      in_specs=[blk(BLK, 128), blk(BLK, 128), blk(BLK, 128), blk(BLK, 128),
                blk(BLK, 16), blk(BLK, 8),
                whole(128, 128), whole(1, 128), whole(128, 128),
                whole(128, 512), whole(128, 512), whole(8, 512),
                whole(1, 512),
                whole(128, 128), whole(128, 128), whole(1, 128),
                whole(128, 64), whole(1, 64), whole(64, 8), whole(1, 8)],
      out_specs=blk(BLK, 8),
      out_shape=jax.ShapeDtypeStruct((B, 8), f32),
  )(ua, ub, uh1, ux, uinv, roh8,
    wl2t, bl2, wr2t, wiha, wihb, wihr8, bih,
    wc1at, wc1bt, bc1, wc2t, bc2, wc3t8, bc3p)


def kernel(x, edge_index, user_ids, current_roles,
           Wl1, bl1, Wr1, Wl2, bl2, Wr2,
           W_ih, W_hh, b_ih, b_hh,
           Wc1, bc1, Wc2, bc2, Wc3, bc3):
  f32 = jnp.float32
  src2d = edge_index[0].reshape(ROWS, CHUNK)
  dst2d = edge_index[1].reshape(ROWS, CHUNK)
  uids2d = user_ids.reshape(NW, CHUNK)

  zeros_f = jnp.zeros((RPT, H), f32)
  zeros_d = jnp.zeros((RPT, 16), f32)
  ones16 = jnp.zeros((CHUNK, 16), f32).at[:, 0].set(1.0)

  s1, d1 = _sc_scatter(x, src2d, dst2d, zeros_f, zeros_d, ones16, True)
  s1a, s1b = s1[:N], s1[NP:NP + N]

  h1, inv16 = _tc_layer1(
      s1a, s1b, d1[:N], d1[NP:NP + N],
      x, Wl1.T, bl1.reshape(1, H), Wr1.T)

  s2, _unused_d2 = _sc_scatter(h1, src2d, dst2d, zeros_f, zeros_d, ones16,
                               False)
  s2a, s2b = s2[:N], s2[NP:NP + N]

  ua, ub, uh1, ux, uinv = _sc_gather(uids2d, s2a, s2b, h1, x, inv16)

  roh8 = jax.nn.one_hot(current_roles, 8, dtype=f32)
  wihr8 = jnp.zeros((8, 4 * H), f32).at[:R].set(W_ih[:, 2 * H:].T)
  wc3t8 = jnp.zeros((H // 2, 8), f32).at[:, :R].set(Wc3.T)
  bc3p = jnp.zeros((1, 8), f32).at[:, :R].set(bc3)

  out8 = _tc_epilogue(
      ua, ub, uh1, ux, uinv, roh8,
      Wl2.T, bl2.reshape(1, H), Wr2.T,
      W_ih[:, :H].T, W_ih[:, H:2 * H].T, wihr8,
      (b_ih + b_hh).reshape(1, 4 * H),
      Wc1[:, :H].T, Wc1[:, H:].T, bc1.reshape(1, H),
      Wc2.T, bc2.reshape(1, H // 2), wc3t8, bc3p)
  return out8[:, :R]


# speedup vs baseline: 3.2923x; 1.0420x over previous
import functools
import jax
import jax.numpy as jnp
from jax import lax
from jax.experimental import pallas as pl
from jax.experimental.pallas import tpu as pltpu
from jax.experimental.pallas import tpu_sc as plsc

N = 10000
E = 320000
D = 128
H = 128
B = 4096
R = 5

NC = 2
NS = 16
NW = NC * NS
CHUNK = 128
ROWS = E // CHUNK
NITER = 80
ROWSP = NW * NITER
NBUF = 2
RPT = 632
NP = NS * RPT
DA = 144

_mesh = functools.partial(
    plsc.VectorSubcoreMesh, core_axis_name="c", subcore_axis_name="s",
    num_cores=NC, num_subcores=NS)


def _sc_scatter(feats, src2d, dst2d, zeros_f, zeros_d, ones16, with_deg):
  f32 = jnp.float32
  NB = 2
  NR = NITER // NB

  @functools.partial(
      pl.kernel,
      out_type=(jax.ShapeDtypeStruct((NC * NP, H), f32),
                jax.ShapeDtypeStruct((NC * NP, 16), f32)),
      mesh=_mesh(),
      scratch_types=[
          [pltpu.VMEM((CHUNK,), jnp.int32) for _ in range(NB)],
          [pltpu.VMEM((CHUNK,), jnp.int32) for _ in range(NB)],
          [pltpu.VMEM((CHUNK, H), f32) for _ in range(NB)],
          pltpu.VMEM((CHUNK, 16), f32),
          pltpu.VMEM_SHARED((NP, H), f32),
          pltpu.VMEM_SHARED((NP, 16), f32),
          [pltpu.SemaphoreType.DMA for _ in range(NB)],
          [pltpu.SemaphoreType.DMA for _ in range(NB)],
          [pltpu.SemaphoreType.DMA for _ in range(NB)],
          [pltpu.SemaphoreType.DMA for _ in range(NB)],
      ],
      compiler_params=pltpu.CompilerParams(use_tc_tiling_on_sc=False),
  )
  def k(feats_h, src_h, dst_h, zf_h, zd_h, ones_h, out_h, dout_h,
        sbuf, dbuf, buf, ones16v, acc, dacc, gsem, ssem, issem, idsem):
    c = lax.axis_index("c")
    s = lax.axis_index("s")
    wid = s * NC + c
    base = 78 * wid + jnp.minimum(wid, 4)
    nreal = 78 + jnp.where(wid < 4, 1, 0)

    pltpu.sync_copy(ones_h, ones16v)
    pltpu.sync_copy(zf_h, acc.at[pl.ds(s * RPT, RPT)])
    pltpu.sync_copy(zd_h, dacc.at[pl.ds(s * RPT, RPT)])
    plsc.subcore_barrier()

    for b in range(NB):
      pltpu.sync_copy(src_h.at[base + b], sbuf[b])
      pltpu.sync_copy(dst_h.at[base + b], dbuf[b])
      pltpu.async_copy(feats_h.at[sbuf[b]], buf[b], gsem[b])

    def body(j, carry):
      for b in range(NB):
        i = j * NB + b
        valid = i < nreal
        nxt_valid = (i + NB) < nreal

        @pl.when(valid)
        def _():
          pltpu.make_async_copy(feats_h.at[sbuf[b]], buf[b], gsem[b]).wait()

          @pl.when(j > 0)
          def _():
            pltpu.make_async_copy(dst_h.at[0], dbuf[b], idsem[b]).wait()

          pltpu.async_copy(buf[b], acc.at[dbuf[b]], ssem[b], add=True)
          if with_deg:
            pltpu.sync_copy(ones16v, dacc.at[dbuf[b]], add=True)

        @pl.when(nxt_valid)
        def _():
          pltpu.async_copy(src_h.at[base + i + NB], sbuf[b], issem[b])

        @pl.when(valid)
        def _():
          pltpu.make_async_copy(buf[b], acc.at[dbuf[b]], ssem[b]).wait()

        @pl.when(nxt_valid)
        def _():
          pltpu.async_copy(dst_h.at[base + i + NB], dbuf[b], idsem[b])
          pltpu.make_async_copy(src_h.at[0], sbuf[b], issem[b]).wait()
          pltpu.async_copy(feats_h.at[sbuf[b]], buf[b], gsem[b])

      return carry

    lax.fori_loop(0, NR, body, 0)
    plsc.subcore_barrier()

    pltpu.sync_copy(acc.at[pl.ds(s * RPT, RPT)],
                    out_h.at[pl.ds(c * NP + s * RPT, RPT)])
    pltpu.sync_copy(dacc.at[pl.ds(s * RPT, RPT)],
                    dout_h.at[pl.ds(c * NP + s * RPT, RPT)])

  return k(feats, src2d, dst2d, zeros_f, zeros_d, ones16)


def _sc_scatter_user(h1, src2d, dst2d, zeros_f, uids2d, x, inv16):
  f32 = jnp.float32
  NB = 2
  NR = NITER // NB
  UPT = B // NS // CHUNK

  @functools.partial(
      pl.kernel,
      out_type=(jax.ShapeDtypeStruct((NC * B, 128), f32),
                jax.ShapeDtypeStruct((B, 128), f32),
                jax.ShapeDtypeStruct((B, 128), f32),
                jax.ShapeDtypeStruct((B, 16), f32)),
      mesh=_mesh(),
      scratch_types=[
          [pltpu.VMEM((CHUNK,), jnp.int32) for _ in range(NB)],
          [pltpu.VMEM((CHUNK,), jnp.int32) for _ in range(NB)],
          [pltpu.VMEM((CHUNK, H), f32) for _ in range(NB)],
          pltpu.VMEM((CHUNK, 16), f32),
          pltpu.VMEM_SHARED((NP, H), f32),
          [pltpu.SemaphoreType.DMA for _ in range(NB)],
          [pltpu.SemaphoreType.DMA for _ in range(NB)],
          [pltpu.SemaphoreType.DMA for _ in range(NB)],
          [pltpu.SemaphoreType.DMA for _ in range(NB)],
      ],
      compiler_params=pltpu.CompilerParams(use_tc_tiling_on_sc=False),
  )
  def k(h1_h, src_h, dst_h, zf_h, uids_h, x_h, inv_h,
        us2_h, uh1_h, ux_h, uinv_h,
        sbuf, dbuf, buf, buf16, acc, gsem, ssem, issem, idsem):
    c = lax.axis_index("c")
    s = lax.axis_index("s")
    wid = s * NC + c
    base = 78 * wid + jnp.minimum(wid, 4)
    nreal = 78 + jnp.where(wid < 4, 1, 0)

    pltpu.sync_copy(zf_h, acc.at[pl.ds(s * RPT, RPT)])
    plsc.subcore_barrier()

    for b in range(NB):
      pltpu.sync_copy(src_h.at[base + b], sbuf[b])
      pltpu.sync_copy(dst_h.at[base + b], dbuf[b])
      pltpu.async_copy(h1_h.at[sbuf[b]], buf[b], gsem[b])

    def body(j, carry):
      for b in range(NB):
        i = j * NB + b
        valid = i < nreal
        nxt_valid = (i + NB) < nreal

        @pl.when(valid)
        def _():
          pltpu.make_async_copy(h1_h.at[sbuf[b]], buf[b], gsem[b]).wait()

          @pl.when(j > 0)
          def _():
            pltpu.make_async_copy(dst_h.at[0], dbuf[b], idsem[b]).wait()

          pltpu.async_copy(buf[b], acc.at[dbuf[b]], ssem[b], add=True)

        @pl.when(nxt_valid)
        def _():
          pltpu.async_copy(src_h.at[base + i + NB], sbuf[b], issem[b])

        @pl.when(valid)
        def _():
          pltpu.make_async_copy(buf[b], acc.at[dbuf[b]], ssem[b]).wait()

        @pl.when(nxt_valid)
        def _():
          pltpu.async_copy(dst_h.at[base + i + NB], dbuf[b], idsem[b])
          pltpu.make_async_copy(src_h.at[0], sbuf[b], issem[b]).wait()
          pltpu.async_copy(h1_h.at[sbuf[b]], buf[b], gsem[b])

      return carry

    lax.fori_loop(0, NR, body, 0)
    plsc.subcore_barrier()

    for t in range(UPT):
      u = s * UPT + t
      pltpu.sync_copy(uids_h.at[u], sbuf[0])
      pltpu.async_copy(acc.at[sbuf[0]], buf[0], gsem[0]).wait()
      pltpu.sync_copy(buf[0], us2_h.at[pl.ds(c * B + u * CHUNK, CHUNK)])

      @pl.when(c == 0)
      def _():
        pltpu.async_copy(h1_h.at[sbuf[0]], buf[1], gsem[1]).wait()
        pltpu.sync_copy(buf[1], uh1_h.at[pl.ds(u * CHUNK, CHUNK)])
        pltpu.async_copy(x_h.at[sbuf[0]], buf[1], gsem[1]).wait()
        pltpu.sync_copy(buf[1], ux_h.at[pl.ds(u * CHUNK, CHUNK)])
        pltpu.async_copy(inv_h.at[sbuf[0]], buf16, gsem[1]).wait()
        pltpu.sync_copy(buf16, uinv_h.at[pl.ds(u * CHUNK, CHUNK)])

  return k(h1, src2d, dst2d, zeros_f, uids2d, x, inv16)


def _tc_layer1(sa, sb, da, db, x, wl1t, bl1, wr1t):
  BLK = 1000
  f32 = jnp.float32

  def body(sa_r, sb_r, da_r, db_r, x_r, wl_r, bl_r, wr_r, h1_r, inv_r):
    s = sa_r[...] + sb_r[...]
    deg = jnp.maximum(da_r[...][:, 0:1] + db_r[...][:, 0:1], 1.0)
    inv = 1.0 / deg
    m = s * inv
    h = (jnp.dot(m, wl_r[...], preferred_element_type=f32) + bl_r[...]
         + jnp.dot(x_r[...], wr_r[...], preferred_element_type=f32))
    h1_r[...] = jnp.maximum(h, 0.0)
    inv_r[...] = jnp.broadcast_to(inv, (BLK, 16))

  blk = lambda m, n: pl.BlockSpec((m, n), lambda i: (i, 0))
  whole = lambda m, n: pl.BlockSpec((m, n), lambda i: (0, 0))
  return pl.pallas_call(
      body,
      grid=(N // BLK,),
      in_specs=[blk(BLK, 128), blk(BLK, 128), blk(BLK, 16), blk(BLK, 16),
                blk(BLK, 128), whole(128, 128), whole(1, 128),
                whole(128, 128)],
      out_specs=[blk(BLK, 128), blk(BLK, 16)],
      out_shape=[jax.ShapeDtypeStruct((N, 128), f32),
                 jax.ShapeDtypeStruct((N, 16), f32)],
  )(sa, sb, da, db, x, wl1t, bl1, wr1t)


def _tc_epilogue(ua, ub, uh1, ux, uinv, roh8,
                 wl2t, bl2, wr2t, wiha, wihb, wihr8, bih,
                 wc1at, wc1bt, bc1, wc2t, bc2, wc3t8, bc3p):
  BLK = 512
  f32 = jnp.float32

  def body(ua_r, ub_r, uh1_r, ux_r, uinv_r, roh_r,
           wl2_r, bl2_r, wr2_r, wiha_r, wihb_r, wihr_r, bih_r,
           wc1a_r, wc1b_r, bc1_r, wc2_r, bc2_r, wc3_r, bc3_r, out_r):
    dot = lambda a, b: jnp.dot(a, b, preferred_element_type=f32)
    m2 = (ua_r[...] + ub_r[...]) * uinv_r[...][:, 0:1]
    ue = dot(m2, wl2_r[...]) + bl2_r[...] + dot(uh1_r[...], wr2_r[...])
    ue = jnp.clip(ue, -10.0, 10.0)
    uf = jnp.clip(ux_r[...], -10.0, 10.0)
    gates = (dot(ue, wiha_r[...]) + dot(uf, wihb_r[...])
             + dot(roh_r[...], wihr_r[...]) + bih_r[...])
    i_g = gates[:, 0:128]
    g_g = gates[:, 256:384]
    o_g = gates[:, 384:512]
    cc = jax.nn.sigmoid(i_g) * jnp.tanh(g_g)
    lo = jnp.clip(jax.nn.sigmoid(o_g) * jnp.tanh(cc), -10.0, 10.0)
    z = jnp.maximum(dot(ue, wc1a_r[...]) + dot(lo, wc1b_r[...]) + bc1_r[...],
                    0.0)
    z2 = jnp.maximum(dot(z, wc2_r[...]) + bc2_r[...], 0.0)
    out_r[...] = dot(z2, wc3_r[...]) + bc3_r[...]

  blk = lambda m, n: pl.BlockSpec((m, n), lambda i: (i, 0))
  ub_spec = pl.BlockSpec((BLK, 128), lambda i: (i + B // BLK, 0))
  whole = lambda m, n: pl.BlockSpec((m, n), lambda i: (0, 0))
  return pl.pallas_call(
      body,
      grid=(B // BLK,),
      in_specs=[blk(BLK, 128), ub_spec, blk(BLK, 128), blk(BLK, 128),
                blk(BLK, 16), blk(BLK, 8),
                whole(128, 128), whole(1, 128), whole(128, 128),
                whole(128, 512), whole(128, 512), whole(8, 512),
                whole(1, 512),
                whole(128, 128), whole(128, 128), whole(1, 128),
                whole(128, 64), whole(1, 64), whole(64, 8), whole(1, 8)],
      out_specs=blk(BLK, 8),
      out_shape=jax.ShapeDtypeStruct((B, 8), f32),
  )(ua, ub, uh1, ux, uinv, roh8,
    wl2t, bl2, wr2t, wiha, wihb, wihr8, bih,
    wc1at, wc1bt, bc1, wc2t, bc2, wc3t8, bc3p)


def kernel(x, edge_index, user_ids, current_roles,
           Wl1, bl1, Wr1, Wl2, bl2, Wr2,
           W_ih, W_hh, b_ih, b_hh,
           Wc1, bc1, Wc2, bc2, Wc3, bc3):
  f32 = jnp.float32
  src2d = edge_index[0].reshape(ROWS, CHUNK)
  dst2d = edge_index[1].reshape(ROWS, CHUNK)
  uids2d = user_ids.reshape(NW, CHUNK)

  zeros_f = jnp.zeros((RPT, H), f32)
  zeros_d = jnp.zeros((RPT, 16), f32)
  ones16 = jnp.zeros((CHUNK, 16), f32).at[:, 0].set(1.0)

  s1, d1 = _sc_scatter(x, src2d, dst2d, zeros_f, zeros_d, ones16, True)
  s1a, s1b = s1[:N], s1[NP:NP + N]

  h1, inv16 = _tc_layer1(
      s1a, s1b, d1[:N], d1[NP:NP + N],
      x, Wl1.T, bl1.reshape(1, H), Wr1.T)

  us2p, uh1, ux, uinv = _sc_scatter_user(
      h1, src2d, dst2d, zeros_f, uids2d, x, inv16)

  roh8 = jax.nn.one_hot(current_roles, 8, dtype=f32)
  wihr8 = jnp.zeros((8, 4 * H), f32).at[:R].set(W_ih[:, 2 * H:].T)
  wc3t8 = jnp.zeros((H // 2, 8), f32).at[:, :R].set(Wc3.T)
  bc3p = jnp.zeros((1, 8), f32).at[:, :R].set(bc3)

  out8 = _tc_epilogue(
      us2p, us2p, uh1, ux, uinv, roh8,
      Wl2.T, bl2.reshape(1, H), Wr2.T,
      W_ih[:, :H].T, W_ih[:, H:2 * H].T, wihr8,
      (b_ih + b_hh).reshape(1, 4 * H),
      Wc1[:, :H].T, Wc1[:, H:].T, bc1.reshape(1, H),
      Wc2.T, bc2.reshape(1, H // 2), wc3t8, bc3p)
  return out8[:, :R]


# speedup vs baseline: 3.5059x; 1.0649x over previous
import functools
import jax
import jax.numpy as jnp
from jax import lax
from jax.experimental import pallas as pl
from jax.experimental.pallas import tpu as pltpu
from jax.experimental.pallas import tpu_sc as plsc

N = 10000
E = 320000
D = 128
H = 128
B = 4096
R = 5

NC = 2
NS = 16
NW = NC * NS
CHUNK = 128
ROWS = E // CHUNK
NITER = 80
ROWSP = NW * NITER
NBUF = 2
RPT = 632
NP = NS * RPT
DA = 144

_mesh = functools.partial(
    plsc.VectorSubcoreMesh, core_axis_name="c", subcore_axis_name="s",
    num_cores=NC, num_subcores=NS)


def _sc_scatter(feats, src2d, dst2d, zeros_f, zeros_d, ones16, with_deg):
  f32 = jnp.float32
  bf16 = jnp.bfloat16
  NB = 2
  NR = NITER // NB

  @functools.partial(
      pl.kernel,
      out_type=(jax.ShapeDtypeStruct((NC * NP, H), bf16),
                jax.ShapeDtypeStruct((NC * NP, 16), f32)),
      mesh=_mesh(),
      scratch_types=[
          [pltpu.VMEM((CHUNK,), jnp.int32) for _ in range(NB)],
          [pltpu.VMEM((CHUNK,), jnp.int32) for _ in range(NB)],
          [pltpu.VMEM((CHUNK, H), bf16) for _ in range(NB)],
          pltpu.VMEM((CHUNK, 16), f32),
          pltpu.VMEM_SHARED((NP, H), bf16),
          pltpu.VMEM_SHARED((NP, 16), f32),
          [pltpu.SemaphoreType.DMA for _ in range(NB)],
          [pltpu.SemaphoreType.DMA for _ in range(NB)],
          [pltpu.SemaphoreType.DMA for _ in range(NB)],
          [pltpu.SemaphoreType.DMA for _ in range(NB)],
      ],
      compiler_params=pltpu.CompilerParams(use_tc_tiling_on_sc=False),
  )
  def k(feats_h, src_h, dst_h, zf_h, zd_h, ones_h, out_h, dout_h,
        sbuf, dbuf, buf, ones16v, acc, dacc, gsem, ssem, issem, idsem):
    c = lax.axis_index("c")
    s = lax.axis_index("s")
    wid = s * NC + c
    base = 78 * wid + jnp.minimum(wid, 4)
    nreal = 78 + jnp.where(wid < 4, 1, 0)

    pltpu.sync_copy(ones_h, ones16v)
    pltpu.sync_copy(zf_h, acc.at[pl.ds(s * RPT, RPT)])
    pltpu.sync_copy(zd_h, dacc.at[pl.ds(s * RPT, RPT)])
    plsc.subcore_barrier()

    for b in range(NB):
      pltpu.sync_copy(src_h.at[base + b], sbuf[b])
      pltpu.sync_copy(dst_h.at[base + b], dbuf[b])
      pltpu.async_copy(feats_h.at[sbuf[b]], buf[b], gsem[b])

    def body(j, carry):
      for b in range(NB):
        i = j * NB + b
        valid = i < nreal
        nxt_valid = (i + NB) < nreal

        @pl.when(valid)
        def _():
          pltpu.make_async_copy(feats_h.at[sbuf[b]], buf[b], gsem[b]).wait()

          @pl.when(j > 0)
          def _():
            pltpu.make_async_copy(dst_h.at[0], dbuf[b], idsem[b]).wait()

          pltpu.async_copy(buf[b], acc.at[dbuf[b]], ssem[b], add=True)
          if with_deg:
            pltpu.sync_copy(ones16v, dacc.at[dbuf[b]], add=True)

        @pl.when(nxt_valid)
        def _():
          pltpu.async_copy(src_h.at[base + i + NB], sbuf[b], issem[b])

        @pl.when(valid)
        def _():
          pltpu.make_async_copy(buf[b], acc.at[dbuf[b]], ssem[b]).wait()

        @pl.when(nxt_valid)
        def _():
          pltpu.async_copy(dst_h.at[base + i + NB], dbuf[b], idsem[b])
          pltpu.make_async_copy(src_h.at[0], sbuf[b], issem[b]).wait()
          pltpu.async_copy(feats_h.at[sbuf[b]], buf[b], gsem[b])

      return carry

    lax.fori_loop(0, NR, body, 0)
    plsc.subcore_barrier()

    pltpu.sync_copy(acc.at[pl.ds(s * RPT, RPT)],
                    out_h.at[pl.ds(c * NP + s * RPT, RPT)])
    pltpu.sync_copy(dacc.at[pl.ds(s * RPT, RPT)],
                    dout_h.at[pl.ds(c * NP + s * RPT, RPT)])

  return k(feats, src2d, dst2d, zeros_f, zeros_d, ones16)


def _sc_scatter_user(h1, src2d, dst2d, zeros_f, uids2d, x, inv16):
  f32 = jnp.float32
  bf16 = jnp.bfloat16
  NB = 2
  NR = NITER // NB
  UPT = B // NS // CHUNK

  @functools.partial(
      pl.kernel,
      out_type=(jax.ShapeDtypeStruct((NC * B, 128), bf16),
                jax.ShapeDtypeStruct((B, 128), bf16),
                jax.ShapeDtypeStruct((B, 128), bf16),
                jax.ShapeDtypeStruct((B, 16), f32)),
      mesh=_mesh(),
      scratch_types=[
          [pltpu.VMEM((CHUNK,), jnp.int32) for _ in range(NB)],
          [pltpu.VMEM((CHUNK,), jnp.int32) for _ in range(NB)],
          [pltpu.VMEM((CHUNK, H), bf16) for _ in range(NB)],
          pltpu.VMEM((CHUNK, 16), f32),
          pltpu.VMEM_SHARED((NP, H), bf16),
          [pltpu.SemaphoreType.DMA for _ in range(NB)],
          [pltpu.SemaphoreType.DMA for _ in range(NB)],
          [pltpu.SemaphoreType.DMA for _ in range(NB)],
          [pltpu.SemaphoreType.DMA for _ in range(NB)],
      ],
      compiler_params=pltpu.CompilerParams(use_tc_tiling_on_sc=False),
  )
  def k(h1_h, src_h, dst_h, zf_h, uids_h, x_h, inv_h,
        us2_h, uh1_h, ux_h, uinv_h,
        sbuf, dbuf, buf, buf16, acc, gsem, ssem, issem, idsem):
    c = lax.axis_index("c")
    s = lax.axis_index("s")
    wid = s * NC + c
    base = 78 * wid + jnp.minimum(wid, 4)
    nreal = 78 + jnp.where(wid < 4, 1, 0)

    pltpu.sync_copy(zf_h, acc.at[pl.ds(s * RPT, RPT)])
    plsc.subcore_barrier()

    for b in range(NB):
      pltpu.sync_copy(src_h.at[base + b], sbuf[b])
      pltpu.sync_copy(dst_h.at[base + b], dbuf[b])
      pltpu.async_copy(h1_h.at[sbuf[b]], buf[b], gsem[b])

    def body(j, carry):
      for b in range(NB):
        i = j * NB + b
        valid = i < nreal
        nxt_valid = (i + NB) < nreal

        @pl.when(valid)
        def _():
          pltpu.make_async_copy(h1_h.at[sbuf[b]], buf[b], gsem[b]).wait()

          @pl.when(j > 0)
          def _():
            pltpu.make_async_copy(dst_h.at[0], dbuf[b], idsem[b]).wait()

          pltpu.async_copy(buf[b], acc.at[dbuf[b]], ssem[b], add=True)

        @pl.when(nxt_valid)
        def _():
          pltpu.async_copy(src_h.at[base + i + NB], sbuf[b], issem[b])

        @pl.when(valid)
        def _():
          pltpu.make_async_copy(buf[b], acc.at[dbuf[b]], ssem[b]).wait()

        @pl.when(nxt_valid)
        def _():
          pltpu.async_copy(dst_h.at[base + i + NB], dbuf[b], idsem[b])
          pltpu.make_async_copy(src_h.at[0], sbuf[b], issem[b]).wait()
          pltpu.async_copy(h1_h.at[sbuf[b]], buf[b], gsem[b])

      return carry

    lax.fori_loop(0, NR, body, 0)
    plsc.subcore_barrier()

    for t in range(UPT):
      u = s * UPT + t
      pltpu.sync_copy(uids_h.at[u], sbuf[0])
      pltpu.async_copy(acc.at[sbuf[0]], buf[0], gsem[0]).wait()
      pltpu.sync_copy(buf[0], us2_h.at[pl.ds(c * B + u * CHUNK, CHUNK)])

      @pl.when(c == 0)
      def _():
        pltpu.async_copy(h1_h.at[sbuf[0]], buf[1], gsem[1]).wait()
        pltpu.sync_copy(buf[1], uh1_h.at[pl.ds(u * CHUNK, CHUNK)])
        pltpu.async_copy(x_h.at[sbuf[0]], buf[1], gsem[1]).wait()
        pltpu.sync_copy(buf[1], ux_h.at[pl.ds(u * CHUNK, CHUNK)])
        pltpu.async_copy(inv_h.at[sbuf[0]], buf16, gsem[1]).wait()
        pltpu.sync_copy(buf16, uinv_h.at[pl.ds(u * CHUNK, CHUNK)])

  return k(h1, src2d, dst2d, zeros_f, uids2d, x, inv16)


def _tc_layer1(sa, sb, da, db, x, wl1t, bl1, wr1t):
  BLK = 1000
  f32 = jnp.float32

  def body(sa_r, sb_r, da_r, db_r, x_r, wl_r, bl_r, wr_r, h1_r, inv_r):
    s = sa_r[...].astype(f32) + sb_r[...].astype(f32)
    deg = jnp.maximum(da_r[...][:, 0:1] + db_r[...][:, 0:1], 1.0)
    inv = 1.0 / deg
    m = s * inv
    h = (jnp.dot(m, wl_r[...], preferred_element_type=f32) + bl_r[...]
         + jnp.dot(x_r[...], wr_r[...], preferred_element_type=f32))
    h1_r[...] = jnp.maximum(h, 0.0).astype(jnp.bfloat16)
    inv_r[...] = jnp.broadcast_to(inv, (BLK, 16))

  blk = lambda m, n: pl.BlockSpec((m, n), lambda i: (i, 0))
  whole = lambda m, n: pl.BlockSpec((m, n), lambda i: (0, 0))
  return pl.pallas_call(
      body,
      grid=(N // BLK,),
      in_specs=[blk(BLK, 128), blk(BLK, 128), blk(BLK, 16), blk(BLK, 16),
                blk(BLK, 128), whole(128, 128), whole(1, 128),
                whole(128, 128)],
      out_specs=[blk(BLK, 128), blk(BLK, 16)],
      out_shape=[jax.ShapeDtypeStruct((N, 128), jnp.bfloat16),
                 jax.ShapeDtypeStruct((N, 16), f32)],
  )(sa, sb, da, db, x, wl1t, bl1, wr1t)


def _tc_epilogue(ua, ub, uh1, ux, uinv, roh8,
                 wl2t, bl2, wr2t, wiha, wihb, wihr8, bih,
                 wc1at, wc1bt, bc1, wc2t, bc2, wc3t8, bc3p):
  BLK = 512
  f32 = jnp.float32

  def body(ua_r, ub_r, uh1_r, ux_r, uinv_r, roh_r,
           wl2_r, bl2_r, wr2_r, wiha_r, wihb_r, wihr_r, bih_r,
           wc1a_r, wc1b_r, bc1_r, wc2_r, bc2_r, wc3_r, bc3_r, out_r):
    dot = lambda a, b: jnp.dot(a, b, preferred_element_type=f32)
    m2 = ((ua_r[...].astype(f32) + ub_r[...].astype(f32))
          * uinv_r[...][:, 0:1])
    ue = (dot(m2, wl2_r[...]) + bl2_r[...]
          + dot(uh1_r[...].astype(f32), wr2_r[...]))
    ue = jnp.clip(ue, -10.0, 10.0)
    uf = jnp.clip(ux_r[...].astype(f32), -10.0, 10.0)
    gates = (dot(ue, wiha_r[...]) + dot(uf, wihb_r[...])
             + dot(roh_r[...], wihr_r[...]) + bih_r[...])
    i_g = gates[:, 0:128]
    g_g = gates[:, 256:384]
    o_g = gates[:, 384:512]
    cc = jax.nn.sigmoid(i_g) * jnp.tanh(g_g)
    lo = jnp.clip(jax.nn.sigmoid(o_g) * jnp.tanh(cc), -10.0, 10.0)
    z = jnp.maximum(dot(ue, wc1a_r[...]) + dot(lo, wc1b_r[...]) + bc1_r[...],
                    0.0)
    z2 = jnp.maximum(dot(z, wc2_r[...]) + bc2_r[...], 0.0)
    out_r[...] = dot(z2, wc3_r[...]) + bc3_r[...]

  blk = lambda m, n: pl.BlockSpec((m, n), lambda i: (i, 0))
  ub_spec = pl.BlockSpec((BLK, 128), lambda i: (i + B // BLK, 0))
  whole = lambda m, n: pl.BlockSpec((m, n), lambda i: (0, 0))
  return pl.pallas_call(
      body,
      grid=(B // BLK,),
      in_specs=[blk(BLK, 128), ub_spec, blk(BLK, 128), blk(BLK, 128),
                blk(BLK, 16), blk(BLK, 8),
                whole(128, 128), whole(1, 128), whole(128, 128),
                whole(128, 512), whole(128, 512), whole(8, 512),
                whole(1, 512),
                whole(128, 128), whole(128, 128), whole(1, 128),
                whole(128, 64), whole(1, 64), whole(64, 8), whole(1, 8)],
      out_specs=blk(BLK, 8),
      out_shape=jax.ShapeDtypeStruct((B, 8), f32),
  )(ua, ub, uh1, ux, uinv, roh8,
    wl2t, bl2, wr2t, wiha, wihb, wihr8, bih,
    wc1at, wc1bt, bc1, wc2t, bc2, wc3t8, bc3p)


def kernel(x, edge_index, user_ids, current_roles,
           Wl1, bl1, Wr1, Wl2, bl2, Wr2,
           W_ih, W_hh, b_ih, b_hh,
           Wc1, bc1, Wc2, bc2, Wc3, bc3):
  f32 = jnp.float32
  src2d = edge_index[0].reshape(ROWS, CHUNK)
  dst2d = edge_index[1].reshape(ROWS, CHUNK)
  uids2d = user_ids.reshape(NW, CHUNK)

  zeros_f = jnp.zeros((RPT, H), jnp.bfloat16)
  zeros_d = jnp.zeros((RPT, 16), f32)
  ones16 = jnp.zeros((CHUNK, 16), f32).at[:, 0].set(1.0)
  xbf = x.astype(jnp.bfloat16)

  s1, d1 = _sc_scatter(xbf, src2d, dst2d, zeros_f, zeros_d, ones16, True)
  s1a, s1b = s1[:N], s1[NP:NP + N]

  h1, inv16 = _tc_layer1(
      s1a, s1b, d1[:N], d1[NP:NP + N],
      x, Wl1.T, bl1.reshape(1, H), Wr1.T)

  us2p, uh1, ux, uinv = _sc_scatter_user(
      h1, src2d, dst2d, zeros_f, uids2d, xbf, inv16)

  roh8 = jax.nn.one_hot(current_roles, 8, dtype=f32)
  wihr8 = jnp.zeros((8, 4 * H), f32).at[:R].set(W_ih[:, 2 * H:].T)
  wc3t8 = jnp.zeros((H // 2, 8), f32).at[:, :R].set(Wc3.T)
  bc3p = jnp.zeros((1, 8), f32).at[:, :R].set(bc3)

  out8 = _tc_epilogue(
      us2p, us2p, uh1, ux, uinv, roh8,
      Wl2.T, bl2.reshape(1, H), Wr2.T,
      W_ih[:, :H].T, W_ih[:, H:2 * H].T, wihr8,
      (b_ih + b_hh).reshape(1, 4 * H),
      Wc1[:, :H].T, Wc1[:, H:].T, bc1.reshape(1, H),
      Wc2.T, bc2.reshape(1, H // 2), wc3t8, bc3p)
  return out8[:, :R]
